# baseline (speedup 1.0000x reference)
import functools

import jax
import jax.numpy as jnp
import numpy as np
from jax import lax
from jax.experimental import pallas as pl
from jax.experimental.pallas import tpu as pltpu
from jax.experimental.pallas import tpu_sc as plsc

N = 10000
NP = 10240
HN = NP // 2
JR = 1024
AR = HN + JR
E = 320000
D = 128
NH = 8
NC = 2
NS = 16
NW = NC * NS
CH = 128
RPW = 80
EP = NW * RPW * CH
ER = NW * RPW
CH2 = 128
RPW2 = EP // (NW * CH2)
ER2 = EP // CH2
BR = 1024
BE = 2048
GN = NP // NS
F32 = jnp.float32


@functools.cache
def _mesh():
    return plsc.VectorSubcoreMesh(
        core_axis_name="c", subcore_axis_name="s",
        num_cores=NC, num_subcores=NS)


_G_np = np.zeros((D, NH), np.float32)
for _d in range(D):
    _G_np[_d, _d // 16] = 1.0
_GT_np = np.ascontiguousarray(_G_np.T)


def _redirect(didx, dred, half):
    nrows, ncols = didx.shape

    def body(r, carry):
        for j in range(ncols // 16):
            v = didx[r, pl.ds(j * 16, 16)]
            junk = HN + (v & (JR - 1))
            if half == 0:
                red = jnp.where(v < HN, v, junk)
            else:
                red = jnp.where(v >= HN, v - HN, junk)
            dred[r, pl.ds(j * 16, 16)] = red
        return carry

    lax.fori_loop(0, nrows, body, 0)


def _zero_acc(acc, zbuf, sid):
    for t in range(AR // NS // CH):
        pltpu.sync_copy(zbuf, acc.at[pl.ds(sid * (AR // NS) + t * CH, CH)])


def _export_acc(acc, rows, out, cid, sid, half):
    for t in range(5):
        r0 = sid * (HN // NS) + t * 64
        pltpu.sync_copy(acc.at[pl.ds(r0, 64)], rows)
        pltpu.sync_copy(rows, out.at[cid, pl.ds(half * HN + r0, 64)])


@functools.cache
def _build_segsum(half):
    return pl.kernel(
        functools.partial(_segsum_body, half),
        out_type=jax.ShapeDtypeStruct((NC, NP, D), F32),
        mesh=_mesh(),
        scratch_types=[
            pltpu.VMEM((RPW2, CH2), jnp.int32),
            pltpu.VMEM((RPW2, CH2), jnp.int32),
            pltpu.VMEM((CH2, D), F32),
            pltpu.VMEM((CH2, D), F32),
            pltpu.VMEM((64, D), F32),
            pltpu.VMEM((CH, D), F32),
            pltpu.SemaphoreType.DMA,
            pltpu.SemaphoreType.DMA,
            pltpu.SemaphoreType.DMA,
            pltpu.SemaphoreType.DMA,
            pltpu.VMEM_SHARED((AR, D), F32),
        ],
    )


def _segsum_body(half, tab, src2, dst2, zeros128, out,
                 sidx, dred, r0, r1, erows, zbuf,
                 g0s, g1s, s0s, s1s, acc):
    cid = lax.axis_index("c")
    sid = lax.axis_index("s")
    w = cid * NS + sid
    rows = [r0, r1]
    gsem = [g0s, g1s]
    ssem = [s0s, s1s]
    rounds = RPW2 // 2
    pltpu.sync_copy(zeros128, zbuf)
    _zero_acc(acc, zbuf, sid)
    pltpu.sync_copy(src2.at[pl.ds(w * RPW2, RPW2)], sidx)
    pltpu.sync_copy(dst2.at[pl.ds(w * RPW2, RPW2)], dred)
    _redirect(dred, dred, half)
    plsc.subcore_barrier()

    for j in range(2):
        pltpu.async_copy(tab.at[sidx.at[j]], rows[j], gsem[j])

    def body(k, carry):
        descs = []
        for j in range(2):
            g = 2 * k + j
            pltpu.make_async_copy(tab.at[sidx.at[g]], rows[j],
                                  gsem[j]).wait()
            descs.append(pltpu.async_copy(rows[j], acc.at[dred.at[g]],
                                          ssem[j], add=True))
        for j in range(2):
            descs[j].wait()

            @pl.when(k < rounds - 1)
            def _(j=j):
                pltpu.async_copy(tab.at[sidx.at[2 * k + 2 + j]], rows[j],
                                 gsem[j])

        return carry

    lax.fori_loop(0, rounds, body, 0)
    plsc.subcore_barrier()
    _export_acc(acc, erows, out, cid, sid, half)


def _sc_segsum_half(tab, src2, dst2, zeros128, half):
    return _build_segsum(half)(tab, src2, dst2, zeros128)


@functools.cache
def _build_scatter_rows(half):
    return pl.kernel(
        functools.partial(_scatter_rows_body, half),
        out_type=jax.ShapeDtypeStruct((NC, NP, D), F32),
        mesh=_mesh(),
        scratch_types=[
            pltpu.VMEM((RPW2, CH2), jnp.int32),
            pltpu.VMEM((CH2, D), F32),
            pltpu.VMEM((CH2, D), F32),
            pltpu.VMEM((64, D), F32),
            pltpu.VMEM((CH, D), F32),
            pltpu.SemaphoreType.DMA,
            pltpu.SemaphoreType.DMA,
            pltpu.SemaphoreType.DMA,
            pltpu.SemaphoreType.DMA,
            pltpu.VMEM_SHARED((AR, D), F32),
        ],
    )


def _scatter_rows_body(half, vals, dst2, zeros128, out,
                       dred, r0, r1, erows, zbuf,
                       g0s, g1s, s0s, s1s, acc):
    cid = lax.axis_index("c")
    sid = lax.axis_index("s")
    w = cid * NS + sid
    rows = [r0, r1]
    gsem = [g0s, g1s]
    ssem = [s0s, s1s]
    rounds = RPW2 // 2
    pltpu.sync_copy(zeros128, zbuf)
    _zero_acc(acc, zbuf, sid)
    pltpu.sync_copy(dst2.at[pl.ds(w * RPW2, RPW2)], dred)
    _redirect(dred, dred, half)
    plsc.subcore_barrier()

    def vsrc(g):
        base = pl.multiple_of((w * RPW2 + g) * CH2, CH2)
        return vals.at[pl.ds(base, CH2)]

    for j in range(2):
        pltpu.async_copy(vsrc(j), rows[j], gsem[j])

    def body(k, carry):
        descs = []
        for j in range(2):
            g = 2 * k + j
            pltpu.make_async_copy(vsrc(g), rows[j], gsem[j]).wait()
            descs.append(pltpu.async_copy(rows[j], acc.at[dred.at[g]],
                                          ssem[j], add=True))
        for j in range(2):
            descs[j].wait()

            @pl.when(k < rounds - 1)
            def _(j=j):
                pltpu.async_copy(vsrc(2 * k + 2 + j), rows[j], gsem[j])

        return carry

    lax.fori_loop(0, rounds, body, 0)
    plsc.subcore_barrier()
    _export_acc(acc, erows, out, cid, sid, half)


def _sc_scatter_rows_half(vals, dst2, zeros128, half):
    return _build_scatter_rows(half)(vals, dst2, zeros128)


@functools.cache
def _build_degree(half):
    return pl.kernel(
        functools.partial(_degree_body, half),
        out_type=jax.ShapeDtypeStruct((NC, NP, D), F32),
        mesh=_mesh(),
        scratch_types=[
            pltpu.VMEM((RPW2, CH2), jnp.int32),
            pltpu.VMEM((CH2, D), F32),
            pltpu.VMEM((64, D), F32),
            pltpu.VMEM((CH, D), F32),
            pltpu.SemaphoreType.DMA,
            pltpu.VMEM_SHARED((AR, D), F32),
        ],
    )


def _degree_body(half, dst2, ones128, zeros128, out,
                 dred, onesb, erows, zbuf, sem0, acc):
    cid = lax.axis_index("c")
    sid = lax.axis_index("s")
    w = cid * NS + sid
    pltpu.sync_copy(zeros128, zbuf)
    _zero_acc(acc, zbuf, sid)
    pltpu.sync_copy(ones128, onesb)
    pltpu.sync_copy(dst2.at[pl.ds(w * RPW2, RPW2)], dred)
    _redirect(dred, dred, half)
    plsc.subcore_barrier()

    def body(k, carry):
        a = pltpu.async_copy(onesb, acc.at[dred.at[2 * k]], sem0, add=True)
        b = pltpu.async_copy(onesb, acc.at[dred.at[2 * k + 1]], sem0,
                             add=True)
        a.wait()
        b.wait()
        return carry

    lax.fori_loop(0, RPW2 // 2, body, 0)
    plsc.subcore_barrier()
    _export_acc(acc, erows, out, cid, sid, half)


def _sc_degree(dst2, ones128, zeros128):
    lo = _build_degree(0)(dst2, ones128, zeros128)
    hi = _build_degree(1)(dst2, ones128, zeros128)
    return jnp.concatenate([lo[:, :HN], hi[:, HN:]], axis=1)


@functools.cache
def _build_gather2():
    return pl.kernel(
        _gather2_body,
        out_type=(jax.ShapeDtypeStruct((EP, D), F32),
                  jax.ShapeDtypeStruct((EP, D), F32)),
        mesh=_mesh(),
        scratch_types=[
            pltpu.VMEM((RPW, CH), jnp.int32),
            pltpu.VMEM((RPW, CH), jnp.int32),
            pltpu.VMEM((CH, D), F32),
            pltpu.VMEM((CH, D), F32),
            pltpu.VMEM((CH, D), F32),
            pltpu.VMEM((CH, D), F32),
            pltpu.SemaphoreType.DMA,
            pltpu.SemaphoreType.DMA,
            pltpu.SemaphoreType.DMA,
            pltpu.SemaphoreType.DMA,
        ],
    )


def _sc_gather2(tab, src2, dst2):
    return _build_gather2()(tab, src2, dst2)


def _gather2_body(tab, src2, dst2, outS, outD, sidx, didx,
                  rS0, rS1, rD0, rD1, semS0, semS1, semD0, semD1):
    cid = lax.axis_index("c")
    sid = lax.axis_index("s")
    w = cid * NS + sid
    pltpu.sync_copy(src2.at[pl.ds(w * RPW, RPW)], sidx)
    pltpu.sync_copy(dst2.at[pl.ds(w * RPW, RPW)], didx)

    def obase(g):
        return pl.multiple_of((w * RPW + g) * CH, CH)

    pltpu.async_copy(tab.at[sidx.at[0]], rS0, semS0)
    pltpu.async_copy(tab.at[didx.at[0]], rD0, semD0)

    def body(k, carry):
        g0 = 2 * k
        pltpu.async_copy(tab.at[sidx.at[g0 + 1]], rS1, semS1)
        pltpu.async_copy(tab.at[didx.at[g0 + 1]], rD1, semD1)
        pltpu.make_async_copy(tab.at[sidx.at[g0]], rS0, semS0).wait()
        pltpu.sync_copy(rS0, outS.at[pl.ds(obase(g0), CH)])
        pltpu.make_async_copy(tab.at[didx.at[g0]], rD0, semD0).wait()
        pltpu.sync_copy(rD0, outD.at[pl.ds(obase(g0), CH)])

        @pl.when(k < RPW // 2 - 1)
        def _():
            pltpu.async_copy(tab.at[sidx.at[g0 + 2]], rS0, semS0)
            pltpu.async_copy(tab.at[didx.at[g0 + 2]], rD0, semD0)

        pltpu.make_async_copy(tab.at[sidx.at[g0 + 1]], rS1, semS1).wait()
        pltpu.sync_copy(rS1, outS.at[pl.ds(obase(g0 + 1), CH)])
        pltpu.make_async_copy(tab.at[didx.at[g0 + 1]], rD1, semD1).wait()
        pltpu.sync_copy(rD1, outD.at[pl.ds(obase(g0 + 1), CH)])
        return carry

    lax.fori_loop(0, RPW // 2, body, 0)


def _ln(x, s, b):
    mu = jnp.mean(x, axis=-1, keepdims=True)
    xc = x - mu
    var = jnp.mean(xc * xc, axis=-1, keepdims=True)
    return xc / jnp.sqrt(var + 1e-5) * s + b


def _elu(x):
    return jnp.where(x > 0, x, jnp.exp(x) - 1.0)


def _dot(a, b):
    return jnp.dot(a, b, preferred_element_type=F32)


_row_spec = pl.BlockSpec((BR, D), lambda i: (i, 0))
_w_spec = pl.BlockSpec((D, D), lambda i: (0, 0))
_v_spec = pl.BlockSpec((1, D), lambda i: (0, 0))


def _p_specs(width):
    return [pl.BlockSpec((1, BR, width), lambda i: (0, i, 0)),
            pl.BlockSpec((1, BR, width), lambda i: (1, i, 0))]


def _pre_body(x, wpre, bpre, s0, b0, hn_ref):
    h = _elu(_dot(x[...], wpre[...]) + bpre[...])
    hn_ref[...] = _ln(h, s0[...], b0[...])


def _tc_pre(x, wpre, bpre, s0, b0):
    return pl.pallas_call(
        _pre_body,
        grid=(NP // BR,),
        in_specs=[_row_spec, _w_spec, _v_spec, _v_spec, _v_spec],
        out_specs=_row_spec,
        out_shape=jax.ShapeDtypeStruct((NP, D), F32),
    )(x, wpre, bpre, s0, b0)


def _conv_tail_body(has_ft, hn, p0, p1, d0, d1, wself, wneigh, bconv,
                    ilns, ilnb, wcsi, bcsi, nlns, nlnb, watt,
                    hn_ref, ft_ref=None):
    deg = jnp.maximum(d0[0][:, :1] + d1[0][:, :1], 1.0)
    neigh = (p0[0] + p1[0]) / deg
    hnv = hn[...]
    h = _dot(hnv, wself[...]) + _dot(neigh, wneigh[...]) + bconv[...] + hnv
    hn2 = _ln(h, ilns[...], ilnb[...])
    h = hn2 + _elu(_dot(hn2, wcsi[...]) + bcsi[...])
    hn_n = _ln(h, nlns[...], nlnb[...])
    hn_ref[...] = hn_n
    if has_ft:
        ft_ref[...] = _dot(hn_n, watt[...])


def _tc_conv_tail(has_ft, hn, pA, pD, wself, wneigh, bconv, ilns, ilnb,
                  wcsi, bcsi, nlns, nlnb, watt):
    n_out = 2 if has_ft else 1
    out_specs = [_row_spec] * n_out
    out_shape = [jax.ShapeDtypeStruct((NP, D), F32)] * n_out
    return pl.pallas_call(
        functools.partial(_conv_tail_body, has_ft),
        grid=(NP // BR,),
        in_specs=[_row_spec] + _p_specs(D) + _p_specs(D)
        + [_w_spec, _w_spec, _v_spec, _v_spec, _v_spec, _w_spec, _v_spec,
           _v_spec, _v_spec, _w_spec],
        out_specs=out_specs if has_ft else out_specs[0],
        out_shape=out_shape if has_ft else out_shape[0],
    )(hn, pA, pA, pD, pD, wself, wneigh, bconv, ilns, ilnb, wcsi, bcsi,
      nlns, nlnb, watt)


def _att_edge_body(ftS, ftD, g_ref, gt_ref, eew_ref, wft_ref):
    fs = ftS[...]
    prod = fs * ftD[...]
    e = _dot(prod, g_ref[...]) * 0.25
    eew = _dot(jnp.exp(e), gt_ref[...])
    eew_ref[...] = eew
    wft_ref[...] = fs * eew


def _tc_att_edge(ftS, ftD, g, gt):
    return pl.pallas_call(
        _att_edge_body,
        grid=(EP // BE,),
        in_specs=[pl.BlockSpec((BE, D), lambda i: (i, 0)),
                  pl.BlockSpec((BE, D), lambda i: (i, 0)),
                  pl.BlockSpec((D, NH), lambda i: (0, 0)),
                  pl.BlockSpec((NH, D), lambda i: (0, 0))],
        out_specs=[pl.BlockSpec((BE, D), lambda i: (i, 0)),
                   pl.BlockSpec((BE, D), lambda i: (i, 0))],
        out_shape=[jax.ShapeDtypeStruct((EP, D), F32),
                   jax.ShapeDtypeStruct((EP, D), F32)],
    )(ftS, ftD, g, gt)


def _att_tail_body(last, a0, a1, e0, e1, hn, whr, bhr, ilns, ilnb,
                   wasi, basi, w_x, b_x, watt, o_ref, ft_ref=None):
    denom = e0[0] + e1[0] + 1e-9
    ha = _elu((a0[0] + a1[0]) / denom)
    h = _dot(ha, whr[...]) + bhr[...] + hn[...]
    hn2 = _ln(h, ilns[...], ilnb[...])
    h = hn2 + _elu(_dot(hn2, wasi[...]) + basi[...])
    if last:
        o_ref[...] = _dot(h, w_x[...]) + b_x[...]
    else:
        hn_n = _ln(h, w_x[...], b_x[...])
        o_ref[...] = hn_n
        ft_ref[...] = _dot(hn_n, watt[...])


def _tc_att_tail(last, pA, pE, hn, whr, bhr, ilns, ilnb, wasi, basi,
                 w_x, b_x, watt):
    n_out = 1 if last else 2
    x_spec = _w_spec if last else _v_spec
    out_specs = [_row_spec] * n_out
    out_shape = [jax.ShapeDtypeStruct((NP, D), F32)] * n_out
    return pl.pallas_call(
        functools.partial(_att_tail_body, last),
        grid=(NP // BR,),
        in_specs=_p_specs(D) + _p_specs(D)
        + [_row_spec,
           _w_spec, _v_spec, _v_spec, _v_spec, _w_spec, _v_spec,
           x_spec, _v_spec, _w_spec],
        out_specs=out_specs if n_out > 1 else out_specs[0],
        out_shape=out_shape if n_out > 1 else out_shape[0],
    )(pA, pA, pE, pE, hn, whr, bhr, ilns, ilnb, wasi, basi, w_x, b_x,
      watt)


def _sc_segsum(tab, src2, dst2, zeros128):
    lo = _sc_segsum_half(tab, src2, dst2, zeros128, 0)
    hi = _sc_segsum_half(tab, src2, dst2, zeros128, 1)
    return jnp.concatenate([lo[:, :HN], hi[:, HN:]], axis=1)


def _sc_scatter_rows(vals, dst2, zeros128):
    lo = _sc_scatter_rows_half(vals, dst2, zeros128, 0)
    hi = _sc_scatter_rows_half(vals, dst2, zeros128, 1)
    return jnp.concatenate([lo[:, :HN], hi[:, HN:]], axis=1)


def kernel(x, edge_index, W_pre, b_pre, conv_ln_s, conv_ln_b, W_self,
           W_neigh, b_conv, conv_iln_s, conv_iln_b, W_csi, b_csi, att_ln_s,
           att_ln_b, W_att, W_hr, b_hr, att_iln_s, att_iln_b, W_asi, b_asi,
           W_post, b_post):
    src = edge_index[0]
    dst = edge_index[1]
    pad_idx = (jnp.arange(EP - E, dtype=jnp.int32) % CH) + N
    src_f = jnp.concatenate([src, pad_idx])
    dst_f = jnp.concatenate([dst, pad_idx])
    src2 = src_f.reshape(ER, CH)
    dst2 = dst_f.reshape(ER, CH)
    src2b = src_f.reshape(ER2, CH2)
    dst2b = dst_f.reshape(ER2, CH2)
    x_p = jnp.pad(x, ((0, NP - N), (0, 0)))

    zeros128 = jnp.zeros((CH, D), F32)
    ones256 = jnp.ones((CH2, D), F32)
    g_mat = jnp.asarray(_G_np)
    gt_mat = jnp.asarray(_GT_np)

    def v(a):
        return a.reshape(1, -1)

    pD = _sc_degree(dst2b, ones256, zeros128)
    hn = _tc_pre(x_p, W_pre, v(b_pre), v(conv_ln_s[0]), v(conv_ln_b[0]))

    for i in range(3):
        pA = _sc_segsum(hn, src2b, dst2b, zeros128)
        has_ft = i == 2
        nlns = v(att_ln_s[0]) if has_ft else v(conv_ln_s[i + 1])
        nlnb = v(att_ln_b[0]) if has_ft else v(conv_ln_b[i + 1])
        outs = _tc_conv_tail(has_ft, hn, pA, pD, W_self[i], W_neigh[i],
                             v(b_conv[i]), v(conv_iln_s[i]),
                             v(conv_iln_b[i]), W_csi[i], v(b_csi[i]),
                             nlns, nlnb, W_att[0])
        if has_ft:
            hn, ft = outs
        else:
            hn = outs

    for i in range(3):
        ftS, ftD = _sc_gather2(ft, src2, dst2)
        eew, wft = _tc_att_edge(ftS, ftD, g_mat, gt_mat)
        pA = _sc_scatter_rows(wft, dst2b, zeros128)
        pE = _sc_scatter_rows(eew, dst2b, zeros128)
        last = i == 2
        w_x = W_post if last else v(att_ln_s[i + 1])
        b_x = v(b_post) if last else v(att_ln_b[i + 1])
        watt = W_att[0] if last else W_att[i + 1]
        outs = _tc_att_tail(last, pA, pE, hn, W_hr[i], v(b_hr[i]),
                            v(att_iln_s[i]), v(att_iln_b[i]), W_asi[i],
                            v(b_asi[i]), w_x, b_x, watt)
        if last:
            out = outs
        else:
            hn, ft = outs

    return out[:N]


# speedup vs baseline: 1.1958x; 1.1958x over previous
import functools

import jax
import jax.numpy as jnp
import numpy as np
from jax import lax
from jax.experimental import pallas as pl
from jax.experimental.pallas import tpu as pltpu
from jax.experimental.pallas import tpu_sc as plsc

N = 10000
NP = 10240
HN = NP // 2
JR = 1024
AR = HN + JR
E = 320000
D = 128
NH = 8
NC = 2
NS = 16
NW = NC * NS
CH = 128
RPW = 80
EP = NW * RPW * CH
ER = NW * RPW
CH2 = 128
RPW2 = EP // (NW * CH2)
ER2 = EP // CH2
BR = 1024
BE = 2048
GN = NP // NS
F32 = jnp.float32


@functools.cache
def _mesh():
    return plsc.VectorSubcoreMesh(
        core_axis_name="c", subcore_axis_name="s",
        num_cores=NC, num_subcores=NS)


_G_np = np.zeros((D, NH), np.float32)
for _d in range(D):
    _G_np[_d, _d // 16] = 1.0
_GT_np = np.ascontiguousarray(_G_np.T)


def _redirect(didx, dred, half):
    nrows, ncols = didx.shape

    def body(r, carry):
        for j in range(ncols // 16):
            v = didx[r, pl.ds(j * 16, 16)]
            junk = HN + (v & (JR - 1))
            if half == 0:
                red = jnp.where(v < HN, v, junk)
            else:
                red = jnp.where(v >= HN, v - HN, junk)
            dred[r, pl.ds(j * 16, 16)] = red
        return carry

    lax.fori_loop(0, nrows, body, 0)


def _zero_acc(acc, zbuf, sid):
    for t in range(AR // NS // CH):
        pltpu.sync_copy(zbuf, acc.at[pl.ds(sid * (AR // NS) + t * CH, CH)])


def _export_acc(acc, rows, out, cid, sid, half):
    for t in range(5):
        r0 = sid * (HN // NS) + t * 64
        pltpu.sync_copy(acc.at[pl.ds(r0, 64)], rows)
        pltpu.sync_copy(rows, out.at[cid, pl.ds(half * HN + r0, 64)])


@functools.cache
def _build_segsum(half):
    return pl.kernel(
        functools.partial(_segsum_body, half),
        out_type=jax.ShapeDtypeStruct((NC, NP, D), F32),
        mesh=_mesh(),
        scratch_types=[
            pltpu.VMEM((RPW2, CH2), jnp.int32),
            pltpu.VMEM((RPW2, CH2), jnp.int32),
            pltpu.VMEM((CH2, D), F32),
            pltpu.VMEM((CH2, D), F32),
            pltpu.VMEM((64, D), F32),
            pltpu.VMEM((CH, D), F32),
            pltpu.SemaphoreType.DMA,
            pltpu.SemaphoreType.DMA,
            pltpu.VMEM_SHARED((AR, D), F32),
        ],
    )


def _segsum_body(half, tab, src2, dst2, zeros128, out,
                 sidx, dred, rows0, rows1, erows, zbuf, sem0, sem1, acc):
    cid = lax.axis_index("c")
    sid = lax.axis_index("s")
    w = cid * NS + sid
    pltpu.sync_copy(zeros128, zbuf)
    _zero_acc(acc, zbuf, sid)
    pltpu.sync_copy(src2.at[pl.ds(w * RPW2, RPW2)], sidx)
    pltpu.sync_copy(dst2.at[pl.ds(w * RPW2, RPW2)], dred)
    _redirect(dred, dred, half)
    plsc.subcore_barrier()

    pltpu.async_copy(tab.at[sidx.at[0]], rows0, sem0)

    def body(k, carry):
        g0 = 2 * k
        pltpu.async_copy(tab.at[sidx.at[g0 + 1]], rows1, sem1)
        pltpu.make_async_copy(tab.at[sidx.at[g0]], rows0, sem0).wait()
        pltpu.sync_copy(rows0, acc.at[dred.at[g0]], add=True)

        @pl.when(k < RPW2 // 2 - 1)
        def _():
            pltpu.async_copy(tab.at[sidx.at[g0 + 2]], rows0, sem0)

        pltpu.make_async_copy(tab.at[sidx.at[g0 + 1]], rows1, sem1).wait()
        pltpu.sync_copy(rows1, acc.at[dred.at[g0 + 1]], add=True)
        return carry

    lax.fori_loop(0, RPW2 // 2, body, 0)
    plsc.subcore_barrier()
    _export_acc(acc, erows, out, cid, sid, half)


def _sc_segsum_half(tab, src2, dst2, zeros128, half):
    return _build_segsum(half)(tab, src2, dst2, zeros128)


@functools.cache
def _build_scatter_rows(half):
    return pl.kernel(
        functools.partial(_scatter_rows_body, half),
        out_type=jax.ShapeDtypeStruct((NC, NP, D), F32),
        mesh=_mesh(),
        scratch_types=[
            pltpu.VMEM((RPW2, CH2), jnp.int32),
            pltpu.VMEM((CH2, D), F32),
            pltpu.VMEM((CH2, D), F32),
            pltpu.VMEM((64, D), F32),
            pltpu.VMEM((CH, D), F32),
            pltpu.SemaphoreType.DMA,
            pltpu.SemaphoreType.DMA,
            pltpu.VMEM_SHARED((AR, D), F32),
        ],
    )


def _scatter_rows_body(half, vals, dst2, zeros128, out,
                       dred, rows0, rows1, erows, zbuf, sem0, sem1, acc):
    cid = lax.axis_index("c")
    sid = lax.axis_index("s")
    w = cid * NS + sid
    pltpu.sync_copy(zeros128, zbuf)
    _zero_acc(acc, zbuf, sid)
    pltpu.sync_copy(dst2.at[pl.ds(w * RPW2, RPW2)], dred)
    _redirect(dred, dred, half)
    plsc.subcore_barrier()

    def vsrc(g):
        base = pl.multiple_of((w * RPW2 + g) * CH2, CH2)
        return vals.at[pl.ds(base, CH2)]

    pltpu.async_copy(vsrc(0), rows0, sem0)

    def body(k, carry):
        g0 = 2 * k
        pltpu.async_copy(vsrc(g0 + 1), rows1, sem1)
        pltpu.make_async_copy(vsrc(g0), rows0, sem0).wait()
        pltpu.sync_copy(rows0, acc.at[dred.at[g0]], add=True)

        @pl.when(k < RPW2 // 2 - 1)
        def _():
            pltpu.async_copy(vsrc(g0 + 2), rows0, sem0)

        pltpu.make_async_copy(vsrc(g0 + 1), rows1, sem1).wait()
        pltpu.sync_copy(rows1, acc.at[dred.at[g0 + 1]], add=True)
        return carry

    lax.fori_loop(0, RPW2 // 2, body, 0)
    plsc.subcore_barrier()
    _export_acc(acc, erows, out, cid, sid, half)


def _sc_scatter_rows_half(vals, dst2, zeros128, half):
    return _build_scatter_rows(half)(vals, dst2, zeros128)


@functools.cache
def _build_degree(half):
    return pl.kernel(
        functools.partial(_degree_body, half),
        out_type=jax.ShapeDtypeStruct((NC, NP, D), F32),
        mesh=_mesh(),
        scratch_types=[
            pltpu.VMEM((RPW2, CH2), jnp.int32),
            pltpu.VMEM((CH2, D), F32),
            pltpu.VMEM((64, D), F32),
            pltpu.VMEM((CH, D), F32),
            pltpu.SemaphoreType.DMA,
            pltpu.VMEM_SHARED((AR, D), F32),
        ],
    )


def _degree_body(half, dst2, ones128, zeros128, out,
                 dred, onesb, erows, zbuf, sem0, acc):
    cid = lax.axis_index("c")
    sid = lax.axis_index("s")
    w = cid * NS + sid
    pltpu.sync_copy(zeros128, zbuf)
    _zero_acc(acc, zbuf, sid)
    pltpu.sync_copy(ones128, onesb)
    pltpu.sync_copy(dst2.at[pl.ds(w * RPW2, RPW2)], dred)
    _redirect(dred, dred, half)
    plsc.subcore_barrier()

    def body(k, carry):
        a = pltpu.async_copy(onesb, acc.at[dred.at[2 * k]], sem0, add=True)
        b = pltpu.async_copy(onesb, acc.at[dred.at[2 * k + 1]], sem0,
                             add=True)
        a.wait()
        b.wait()
        return carry

    lax.fori_loop(0, RPW2 // 2, body, 0)
    plsc.subcore_barrier()
    _export_acc(acc, erows, out, cid, sid, half)


def _sc_degree(dst2, ones128, zeros128):
    lo = _build_degree(0)(dst2, ones128, zeros128)
    hi = _build_degree(1)(dst2, ones128, zeros128)
    return jnp.concatenate([lo[:, :HN], hi[:, HN:]], axis=1)


@functools.cache
def _build_gather2():
    return pl.kernel(
        _gather2_body,
        out_type=(jax.ShapeDtypeStruct((EP, D), F32),
                  jax.ShapeDtypeStruct((EP, D), F32)),
        mesh=_mesh(),
        scratch_types=[
            pltpu.VMEM((RPW, CH), jnp.int32),
            pltpu.VMEM((RPW, CH), jnp.int32),
            pltpu.VMEM((CH, D), F32),
            pltpu.VMEM((CH, D), F32),
            pltpu.VMEM((CH, D), F32),
            pltpu.VMEM((CH, D), F32),
            pltpu.SemaphoreType.DMA,
            pltpu.SemaphoreType.DMA,
            pltpu.SemaphoreType.DMA,
            pltpu.SemaphoreType.DMA,
        ],
    )


def _sc_gather2(tab, src2, dst2):
    return _build_gather2()(tab, src2, dst2)


def _gather2_body(tab, src2, dst2, outS, outD, sidx, didx,
                  rS0, rS1, rD0, rD1, semS0, semS1, semD0, semD1):
    cid = lax.axis_index("c")
    sid = lax.axis_index("s")
    w = cid * NS + sid
    pltpu.sync_copy(src2.at[pl.ds(w * RPW, RPW)], sidx)
    pltpu.sync_copy(dst2.at[pl.ds(w * RPW, RPW)], didx)

    def obase(g):
        return pl.multiple_of((w * RPW + g) * CH, CH)

    pltpu.async_copy(tab.at[sidx.at[0]], rS0, semS0)
    pltpu.async_copy(tab.at[didx.at[0]], rD0, semD0)

    def body(k, carry):
        g0 = 2 * k
        pltpu.async_copy(tab.at[sidx.at[g0 + 1]], rS1, semS1)
        pltpu.async_copy(tab.at[didx.at[g0 + 1]], rD1, semD1)
        pltpu.make_async_copy(tab.at[sidx.at[g0]], rS0, semS0).wait()
        pltpu.sync_copy(rS0, outS.at[pl.ds(obase(g0), CH)])
        pltpu.make_async_copy(tab.at[didx.at[g0]], rD0, semD0).wait()
        pltpu.sync_copy(rD0, outD.at[pl.ds(obase(g0), CH)])

        @pl.when(k < RPW // 2 - 1)
        def _():
            pltpu.async_copy(tab.at[sidx.at[g0 + 2]], rS0, semS0)
            pltpu.async_copy(tab.at[didx.at[g0 + 2]], rD0, semD0)

        pltpu.make_async_copy(tab.at[sidx.at[g0 + 1]], rS1, semS1).wait()
        pltpu.sync_copy(rS1, outS.at[pl.ds(obase(g0 + 1), CH)])
        pltpu.make_async_copy(tab.at[didx.at[g0 + 1]], rD1, semD1).wait()
        pltpu.sync_copy(rD1, outD.at[pl.ds(obase(g0 + 1), CH)])
        return carry

    lax.fori_loop(0, RPW // 2, body, 0)


def _ln(x, s, b):
    mu = jnp.mean(x, axis=-1, keepdims=True)
    xc = x - mu
    var = jnp.mean(xc * xc, axis=-1, keepdims=True)
    return xc / jnp.sqrt(var + 1e-5) * s + b


def _elu(x):
    return jnp.where(x > 0, x, jnp.exp(x) - 1.0)


def _dot(a, b):
    return jnp.dot(a, b, preferred_element_type=F32)


_row_spec = pl.BlockSpec((BR, D), lambda i: (i, 0))
_w_spec = pl.BlockSpec((D, D), lambda i: (0, 0))
_v_spec = pl.BlockSpec((1, D), lambda i: (0, 0))


def _p_specs(width):
    return [pl.BlockSpec((1, BR, width), lambda i: (0, i, 0)),
            pl.BlockSpec((1, BR, width), lambda i: (1, i, 0))]


def _pre_body(x, wpre, bpre, s0, b0, hn_ref):
    h = _elu(_dot(x[...], wpre[...]) + bpre[...])
    hn_ref[...] = _ln(h, s0[...], b0[...])


def _tc_pre(x, wpre, bpre, s0, b0):
    return pl.pallas_call(
        _pre_body,
        grid=(NP // BR,),
        in_specs=[_row_spec, _w_spec, _v_spec, _v_spec, _v_spec],
        out_specs=_row_spec,
        out_shape=jax.ShapeDtypeStruct((NP, D), F32),
    )(x, wpre, bpre, s0, b0)


def _conv_tail_body(has_ft, hn, p0, p1, d0, d1, wself, wneigh, bconv,
                    ilns, ilnb, wcsi, bcsi, nlns, nlnb, watt,
                    hn_ref, ft_ref=None):
    deg = jnp.maximum(d0[0][:, :1] + d1[0][:, :1], 1.0)
    neigh = (p0[0] + p1[0]) / deg
    hnv = hn[...]
    h = _dot(hnv, wself[...]) + _dot(neigh, wneigh[...]) + bconv[...] + hnv
    hn2 = _ln(h, ilns[...], ilnb[...])
    h = hn2 + _elu(_dot(hn2, wcsi[...]) + bcsi[...])
    hn_n = _ln(h, nlns[...], nlnb[...])
    hn_ref[...] = hn_n
    if has_ft:
        ft_ref[...] = _dot(hn_n, watt[...])


def _tc_conv_tail(has_ft, hn, pA, pD, wself, wneigh, bconv, ilns, ilnb,
                  wcsi, bcsi, nlns, nlnb, watt):
    n_out = 2 if has_ft else 1
    out_specs = [_row_spec] * n_out
    out_shape = [jax.ShapeDtypeStruct((NP, D), F32)] * n_out
    return pl.pallas_call(
        functools.partial(_conv_tail_body, has_ft),
        grid=(NP // BR,),
        in_specs=[_row_spec] + _p_specs(D) + _p_specs(D)
        + [_w_spec, _w_spec, _v_spec, _v_spec, _v_spec, _w_spec, _v_spec,
           _v_spec, _v_spec, _w_spec],
        out_specs=out_specs if has_ft else out_specs[0],
        out_shape=out_shape if has_ft else out_shape[0],
    )(hn, pA, pA, pD, pD, wself, wneigh, bconv, ilns, ilnb, wcsi, bcsi,
      nlns, nlnb, watt)


def _att_edge_body(ftS, ftD, g_ref, gt_ref, eew_ref, wft_ref):
    fs = ftS[...]
    prod = fs * ftD[...]
    e = _dot(prod, g_ref[...]) * 0.25
    eew = _dot(jnp.exp(e), gt_ref[...])
    eew_ref[...] = eew
    wft_ref[...] = fs * eew


def _tc_att_edge(ftS, ftD, g, gt):
    return pl.pallas_call(
        _att_edge_body,
        grid=(EP // BE,),
        in_specs=[pl.BlockSpec((BE, D), lambda i: (i, 0)),
                  pl.BlockSpec((BE, D), lambda i: (i, 0)),
                  pl.BlockSpec((D, NH), lambda i: (0, 0)),
                  pl.BlockSpec((NH, D), lambda i: (0, 0))],
        out_specs=[pl.BlockSpec((BE, D), lambda i: (i, 0)),
                   pl.BlockSpec((BE, D), lambda i: (i, 0))],
        out_shape=[jax.ShapeDtypeStruct((EP, D), F32),
                   jax.ShapeDtypeStruct((EP, D), F32)],
    )(ftS, ftD, g, gt)


def _att_tail_body(last, a0, a1, e0, e1, hn, whr, bhr, ilns, ilnb,
                   wasi, basi, w_x, b_x, watt, o_ref, ft_ref=None):
    denom = e0[0] + e1[0] + 1e-9
    ha = _elu((a0[0] + a1[0]) / denom)
    h = _dot(ha, whr[...]) + bhr[...] + hn[...]
    hn2 = _ln(h, ilns[...], ilnb[...])
    h = hn2 + _elu(_dot(hn2, wasi[...]) + basi[...])
    if last:
        o_ref[...] = _dot(h, w_x[...]) + b_x[...]
    else:
        hn_n = _ln(h, w_x[...], b_x[...])
        o_ref[...] = hn_n
        ft_ref[...] = _dot(hn_n, watt[...])


def _tc_att_tail(last, pA, pE, hn, whr, bhr, ilns, ilnb, wasi, basi,
                 w_x, b_x, watt):
    n_out = 1 if last else 2
    x_spec = _w_spec if last else _v_spec
    out_specs = [_row_spec] * n_out
    out_shape = [jax.ShapeDtypeStruct((NP, D), F32)] * n_out
    return pl.pallas_call(
        functools.partial(_att_tail_body, last),
        grid=(NP // BR,),
        in_specs=_p_specs(D) + _p_specs(D)
        + [_row_spec,
           _w_spec, _v_spec, _v_spec, _v_spec, _w_spec, _v_spec,
           x_spec, _v_spec, _w_spec],
        out_specs=out_specs if n_out > 1 else out_specs[0],
        out_shape=out_shape if n_out > 1 else out_shape[0],
    )(pA, pA, pE, pE, hn, whr, bhr, ilns, ilnb, wasi, basi, w_x, b_x,
      watt)


def _sc_segsum(tab, src2, dst2, zeros128):
    lo = _sc_segsum_half(tab, src2, dst2, zeros128, 0)
    hi = _sc_segsum_half(tab, src2, dst2, zeros128, 1)
    return jnp.concatenate([lo[:, :HN], hi[:, HN:]], axis=1)


def _sc_scatter_rows(vals, dst2, zeros128):
    lo = _sc_scatter_rows_half(vals, dst2, zeros128, 0)
    hi = _sc_scatter_rows_half(vals, dst2, zeros128, 1)
    return jnp.concatenate([lo[:, :HN], hi[:, HN:]], axis=1)


def kernel(x, edge_index, W_pre, b_pre, conv_ln_s, conv_ln_b, W_self,
           W_neigh, b_conv, conv_iln_s, conv_iln_b, W_csi, b_csi, att_ln_s,
           att_ln_b, W_att, W_hr, b_hr, att_iln_s, att_iln_b, W_asi, b_asi,
           W_post, b_post):
    src = edge_index[0]
    dst = edge_index[1]
    pad_idx = (jnp.arange(EP - E, dtype=jnp.int32) % CH) + N
    src_f = jnp.concatenate([src, pad_idx])
    dst_f = jnp.concatenate([dst, pad_idx])
    src2 = src_f.reshape(ER, CH)
    dst2 = dst_f.reshape(ER, CH)
    src2b = src_f.reshape(ER2, CH2)
    dst2b = dst_f.reshape(ER2, CH2)
    x_p = jnp.pad(x, ((0, NP - N), (0, 0)))

    zeros128 = jnp.zeros((CH, D), F32)
    ones256 = jnp.ones((CH2, D), F32)
    g_mat = jnp.asarray(_G_np)
    gt_mat = jnp.asarray(_GT_np)

    def v(a):
        return a.reshape(1, -1)

    pD = _sc_degree(dst2b, ones256, zeros128)
    hn = _tc_pre(x_p, W_pre, v(b_pre), v(conv_ln_s[0]), v(conv_ln_b[0]))

    for i in range(3):
        pA = _sc_segsum(hn, src2b, dst2b, zeros128)
        has_ft = i == 2
        nlns = v(att_ln_s[0]) if has_ft else v(conv_ln_s[i + 1])
        nlnb = v(att_ln_b[0]) if has_ft else v(conv_ln_b[i + 1])
        outs = _tc_conv_tail(has_ft, hn, pA, pD, W_self[i], W_neigh[i],
                             v(b_conv[i]), v(conv_iln_s[i]),
                             v(conv_iln_b[i]), W_csi[i], v(b_csi[i]),
                             nlns, nlnb, W_att[0])
        if has_ft:
            hn, ft = outs
        else:
            hn = outs

    for i in range(3):
        ftS, ftD = _sc_gather2(ft, src2, dst2)
        eew, wft = _tc_att_edge(ftS, ftD, g_mat, gt_mat)
        pA = _sc_scatter_rows(wft, dst2b, zeros128)
        pE = _sc_scatter_rows(eew, dst2b, zeros128)
        last = i == 2
        w_x = W_post if last else v(att_ln_s[i + 1])
        b_x = v(b_post) if last else v(att_ln_b[i + 1])
        watt = W_att[0] if last else W_att[i + 1]
        outs = _tc_att_tail(last, pA, pE, hn, W_hr[i], v(b_hr[i]),
                            v(att_iln_s[i]), v(att_iln_b[i]), W_asi[i],
                            v(b_asi[i]), w_x, b_x, watt)
        if last:
            out = outs
        else:
            hn, ft = outs

    return out[:N]


# speedup vs baseline: 1.2270x; 1.0261x over previous
import functools

import jax
import jax.numpy as jnp
import numpy as np
from jax import lax
from jax.experimental import pallas as pl
from jax.experimental.pallas import tpu as pltpu
from jax.experimental.pallas import tpu_sc as plsc

N = 10000
NP = 10240
HN = NP // 2
JR = 1024
AR = HN + JR
E = 320000
D = 128
NH = 8
NC = 2
NS = 16
NW = NC * NS
CH = 128
RPW = 80
EP = NW * RPW * CH
ER = NW * RPW
CH2 = 128
RPW2 = EP // (NW * CH2)
ER2 = EP // CH2
BR = 1024
BE = 4096
GN = NP // NS
F32 = jnp.float32


@functools.cache
def _mesh():
    return plsc.VectorSubcoreMesh(
        core_axis_name="c", subcore_axis_name="s",
        num_cores=NC, num_subcores=NS)


_G_np = np.zeros((D, NH), np.float32)
for _d in range(D):
    _G_np[_d, _d // 16] = 1.0
_GT_np = np.ascontiguousarray(_G_np.T)


def _redirect(didx, dred, half):
    nrows, ncols = didx.shape

    def body(r, carry):
        for j in range(ncols // 16):
            v = didx[r, pl.ds(j * 16, 16)]
            junk = HN + (v & (JR - 1))
            if half == 0:
                red = jnp.where(v < HN, v, junk)
            else:
                red = jnp.where(v >= HN, v - HN, junk)
            dred[r, pl.ds(j * 16, 16)] = red
        return carry

    lax.fori_loop(0, nrows, body, 0)


def _zero_acc(acc, zbuf, sid):
    for t in range(AR // NS // CH):
        pltpu.sync_copy(zbuf, acc.at[pl.ds(sid * (AR // NS) + t * CH, CH)])


def _export_acc(acc, rows, out, cid, sid, half):
    for t in range(5):
        r0 = sid * (HN // NS) + t * 64
        pltpu.sync_copy(acc.at[pl.ds(r0, 64)], rows)
        pltpu.sync_copy(rows, out.at[cid, pl.ds(half * HN + r0, 64)])


@functools.cache
def _build_segsum(half):
    return pl.kernel(
        functools.partial(_segsum_body, half),
        out_type=jax.ShapeDtypeStruct((NC, NP, D), F32),
        mesh=_mesh(),
        scratch_types=[
            pltpu.VMEM((RPW2, CH2), jnp.int32),
            pltpu.VMEM((RPW2, CH2), jnp.int32),
            pltpu.VMEM((CH2, D), F32),
            pltpu.VMEM((CH2, D), F32),
            pltpu.VMEM((64, D), F32),
            pltpu.VMEM((CH, D), F32),
            pltpu.SemaphoreType.DMA,
            pltpu.SemaphoreType.DMA,
            pltpu.VMEM_SHARED((AR, D), F32),
        ],
    )


def _segsum_body(half, tab, src2, dst2, zeros128, out,
                 sidx, dred, rows0, rows1, erows, zbuf, sem0, sem1, acc):
    cid = lax.axis_index("c")
    sid = lax.axis_index("s")
    w = cid * NS + sid
    pltpu.sync_copy(zeros128, zbuf)
    _zero_acc(acc, zbuf, sid)
    pltpu.sync_copy(src2.at[pl.ds(w * RPW2, RPW2)], sidx)
    pltpu.sync_copy(dst2.at[pl.ds(w * RPW2, RPW2)], dred)
    _redirect(dred, dred, half)
    plsc.subcore_barrier()

    pltpu.async_copy(tab.at[sidx.at[0]], rows0, sem0)

    def body(k, carry):
        g0 = 2 * k
        pltpu.async_copy(tab.at[sidx.at[g0 + 1]], rows1, sem1)
        pltpu.make_async_copy(tab.at[sidx.at[g0]], rows0, sem0).wait()
        pltpu.sync_copy(rows0, acc.at[dred.at[g0]], add=True)

        @pl.when(k < RPW2 // 2 - 1)
        def _():
            pltpu.async_copy(tab.at[sidx.at[g0 + 2]], rows0, sem0)

        pltpu.make_async_copy(tab.at[sidx.at[g0 + 1]], rows1, sem1).wait()
        pltpu.sync_copy(rows1, acc.at[dred.at[g0 + 1]], add=True)
        return carry

    lax.fori_loop(0, RPW2 // 2, body, 0)
    plsc.subcore_barrier()
    _export_acc(acc, erows, out, cid, sid, half)


def _sc_segsum_half(tab, src2, dst2, zeros128, half):
    return _build_segsum(half)(tab, src2, dst2, zeros128)


@functools.cache
def _build_scatter_rows(half):
    return pl.kernel(
        functools.partial(_scatter_rows_body, half),
        out_type=jax.ShapeDtypeStruct((NC, NP, D), F32),
        mesh=_mesh(),
        scratch_types=[
            pltpu.VMEM((RPW2, CH2), jnp.int32),
            pltpu.VMEM((CH2, D), F32),
            pltpu.VMEM((CH2, D), F32),
            pltpu.VMEM((64, D), F32),
            pltpu.VMEM((CH, D), F32),
            pltpu.SemaphoreType.DMA,
            pltpu.SemaphoreType.DMA,
            pltpu.VMEM_SHARED((AR, D), F32),
        ],
    )


def _scatter_rows_body(half, vals, dst2, zeros128, out,
                       dred, rows0, rows1, erows, zbuf, sem0, sem1, acc):
    cid = lax.axis_index("c")
    sid = lax.axis_index("s")
    w = cid * NS + sid
    pltpu.sync_copy(zeros128, zbuf)
    _zero_acc(acc, zbuf, sid)
    pltpu.sync_copy(dst2.at[pl.ds(w * RPW2, RPW2)], dred)
    _redirect(dred, dred, half)
    plsc.subcore_barrier()

    def vsrc(g):
        base = pl.multiple_of((w * RPW2 + g) * CH2, CH2)
        return vals.at[pl.ds(base, CH2)]

    pltpu.async_copy(vsrc(0), rows0, sem0)

    def body(k, carry):
        g0 = 2 * k
        pltpu.async_copy(vsrc(g0 + 1), rows1, sem1)
        pltpu.make_async_copy(vsrc(g0), rows0, sem0).wait()
        pltpu.sync_copy(rows0, acc.at[dred.at[g0]], add=True)

        @pl.when(k < RPW2 // 2 - 1)
        def _():
            pltpu.async_copy(vsrc(g0 + 2), rows0, sem0)

        pltpu.make_async_copy(vsrc(g0 + 1), rows1, sem1).wait()
        pltpu.sync_copy(rows1, acc.at[dred.at[g0 + 1]], add=True)
        return carry

    lax.fori_loop(0, RPW2 // 2, body, 0)
    plsc.subcore_barrier()
    _export_acc(acc, erows, out, cid, sid, half)


def _sc_scatter_rows_half(vals, dst2, zeros128, half):
    return _build_scatter_rows(half)(vals, dst2, zeros128)


@functools.cache
def _build_degree(half):
    return pl.kernel(
        functools.partial(_degree_body, half),
        out_type=jax.ShapeDtypeStruct((NC, NP, D), F32),
        mesh=_mesh(),
        scratch_types=[
            pltpu.VMEM((RPW2, CH2), jnp.int32),
            pltpu.VMEM((CH2, D), F32),
            pltpu.VMEM((64, D), F32),
            pltpu.VMEM((CH, D), F32),
            pltpu.SemaphoreType.DMA,
            pltpu.VMEM_SHARED((AR, D), F32),
        ],
    )


def _degree_body(half, dst2, ones128, zeros128, out,
                 dred, onesb, erows, zbuf, sem0, acc):
    cid = lax.axis_index("c")
    sid = lax.axis_index("s")
    w = cid * NS + sid
    pltpu.sync_copy(zeros128, zbuf)
    _zero_acc(acc, zbuf, sid)
    pltpu.sync_copy(ones128, onesb)
    pltpu.sync_copy(dst2.at[pl.ds(w * RPW2, RPW2)], dred)
    _redirect(dred, dred, half)
    plsc.subcore_barrier()

    def body(k, carry):
        a = pltpu.async_copy(onesb, acc.at[dred.at[2 * k]], sem0, add=True)
        b = pltpu.async_copy(onesb, acc.at[dred.at[2 * k + 1]], sem0,
                             add=True)
        a.wait()
        b.wait()
        return carry

    lax.fori_loop(0, RPW2 // 2, body, 0)
    plsc.subcore_barrier()
    _export_acc(acc, erows, out, cid, sid, half)


def _sc_degree(dst2, ones128, zeros128):
    lo = _build_degree(0)(dst2, ones128, zeros128)
    hi = _build_degree(1)(dst2, ones128, zeros128)
    return jnp.concatenate([lo[:, :HN], hi[:, HN:]], axis=1)


@functools.cache
def _build_gather2():
    return pl.kernel(
        _gather2_body,
        out_type=(jax.ShapeDtypeStruct((EP, D), F32),
                  jax.ShapeDtypeStruct((EP, D), F32)),
        mesh=_mesh(),
        scratch_types=[
            pltpu.VMEM((RPW, CH), jnp.int32),
            pltpu.VMEM((RPW, CH), jnp.int32),
            pltpu.VMEM((CH, D), F32),
            pltpu.VMEM((CH, D), F32),
            pltpu.VMEM((CH, D), F32),
            pltpu.VMEM((CH, D), F32),
            pltpu.SemaphoreType.DMA,
            pltpu.SemaphoreType.DMA,
            pltpu.SemaphoreType.DMA,
            pltpu.SemaphoreType.DMA,
        ],
    )


def _sc_gather2(tab, src2, dst2):
    return _build_gather2()(tab, src2, dst2)


def _gather2_body(tab, src2, dst2, outS, outD, sidx, didx,
                  rS0, rS1, rD0, rD1, semS0, semS1, semD0, semD1):
    cid = lax.axis_index("c")
    sid = lax.axis_index("s")
    w = cid * NS + sid
    pltpu.sync_copy(src2.at[pl.ds(w * RPW, RPW)], sidx)
    pltpu.sync_copy(dst2.at[pl.ds(w * RPW, RPW)], didx)

    def obase(g):
        return pl.multiple_of((w * RPW + g) * CH, CH)

    pltpu.async_copy(tab.at[sidx.at[0]], rS0, semS0)
    pltpu.async_copy(tab.at[didx.at[0]], rD0, semD0)

    def body(k, carry):
        g0 = 2 * k
        pltpu.async_copy(tab.at[sidx.at[g0 + 1]], rS1, semS1)
        pltpu.async_copy(tab.at[didx.at[g0 + 1]], rD1, semD1)
        pltpu.make_async_copy(tab.at[sidx.at[g0]], rS0, semS0).wait()
        pltpu.sync_copy(rS0, outS.at[pl.ds(obase(g0), CH)])
        pltpu.make_async_copy(tab.at[didx.at[g0]], rD0, semD0).wait()
        pltpu.sync_copy(rD0, outD.at[pl.ds(obase(g0), CH)])

        @pl.when(k < RPW // 2 - 1)
        def _():
            pltpu.async_copy(tab.at[sidx.at[g0 + 2]], rS0, semS0)
            pltpu.async_copy(tab.at[didx.at[g0 + 2]], rD0, semD0)

        pltpu.make_async_copy(tab.at[sidx.at[g0 + 1]], rS1, semS1).wait()
        pltpu.sync_copy(rS1, outS.at[pl.ds(obase(g0 + 1), CH)])
        pltpu.make_async_copy(tab.at[didx.at[g0 + 1]], rD1, semD1).wait()
        pltpu.sync_copy(rD1, outD.at[pl.ds(obase(g0 + 1), CH)])
        return carry

    lax.fori_loop(0, RPW // 2, body, 0)


def _ln(x, s, b):
    mu = jnp.mean(x, axis=-1, keepdims=True)
    xc = x - mu
    var = jnp.mean(xc * xc, axis=-1, keepdims=True)
    return xc / jnp.sqrt(var + 1e-5) * s + b


def _elu(x):
    return jnp.where(x > 0, x, jnp.exp(x) - 1.0)


def _dot(a, b):
    return jnp.dot(a, b, preferred_element_type=F32)


_row_spec = pl.BlockSpec((BR, D), lambda i: (i, 0))
_w_spec = pl.BlockSpec((D, D), lambda i: (0, 0))
_v_spec = pl.BlockSpec((1, D), lambda i: (0, 0))


def _p_specs(width):
    return [pl.BlockSpec((1, BR, width), lambda i: (0, i, 0)),
            pl.BlockSpec((1, BR, width), lambda i: (1, i, 0))]


def _pre_body(x, wpre, bpre, s0, b0, hn_ref):
    h = _elu(_dot(x[...], wpre[...]) + bpre[...])
    hn_ref[...] = _ln(h, s0[...], b0[...])


def _tc_pre(x, wpre, bpre, s0, b0):
    return pl.pallas_call(
        _pre_body,
        grid=(NP // BR,),
        in_specs=[_row_spec, _w_spec, _v_spec, _v_spec, _v_spec],
        out_specs=_row_spec,
        out_shape=jax.ShapeDtypeStruct((NP, D), F32),
    )(x, wpre, bpre, s0, b0)


def _conv_tail_body(has_ft, hn, p0, p1, d0, d1, wself, wneigh, bconv,
                    ilns, ilnb, wcsi, bcsi, nlns, nlnb, watt,
                    hn_ref, ft_ref=None):
    deg = jnp.maximum(d0[0][:, :1] + d1[0][:, :1], 1.0)
    neigh = (p0[0] + p1[0]) / deg
    hnv = hn[...]
    h = _dot(hnv, wself[...]) + _dot(neigh, wneigh[...]) + bconv[...] + hnv
    hn2 = _ln(h, ilns[...], ilnb[...])
    h = hn2 + _elu(_dot(hn2, wcsi[...]) + bcsi[...])
    hn_n = _ln(h, nlns[...], nlnb[...])
    hn_ref[...] = hn_n
    if has_ft:
        ft_ref[...] = _dot(hn_n, watt[...])


def _tc_conv_tail(has_ft, hn, pA, pD, wself, wneigh, bconv, ilns, ilnb,
                  wcsi, bcsi, nlns, nlnb, watt):
    n_out = 2 if has_ft else 1
    out_specs = [_row_spec] * n_out
    out_shape = [jax.ShapeDtypeStruct((NP, D), F32)] * n_out
    return pl.pallas_call(
        functools.partial(_conv_tail_body, has_ft),
        grid=(NP // BR,),
        in_specs=[_row_spec] + _p_specs(D) + _p_specs(D)
        + [_w_spec, _w_spec, _v_spec, _v_spec, _v_spec, _w_spec, _v_spec,
           _v_spec, _v_spec, _w_spec],
        out_specs=out_specs if has_ft else out_specs[0],
        out_shape=out_shape if has_ft else out_shape[0],
    )(hn, pA, pA, pD, pD, wself, wneigh, bconv, ilns, ilnb, wcsi, bcsi,
      nlns, nlnb, watt)


def _att_edge_body(ftS, ftD, g_ref, gt_ref, eew_ref, wft_ref):
    fs = ftS[...]
    prod = fs * ftD[...]
    e = _dot(prod, g_ref[...]) * 0.25
    eew = _dot(jnp.exp(e), gt_ref[...])
    eew_ref[...] = eew
    wft_ref[...] = fs * eew


def _tc_att_edge(ftS, ftD, g, gt):
    return pl.pallas_call(
        _att_edge_body,
        grid=(EP // BE,),
        in_specs=[pl.BlockSpec((BE, D), lambda i: (i, 0)),
                  pl.BlockSpec((BE, D), lambda i: (i, 0)),
                  pl.BlockSpec((D, NH), lambda i: (0, 0)),
                  pl.BlockSpec((NH, D), lambda i: (0, 0))],
        out_specs=[pl.BlockSpec((BE, D), lambda i: (i, 0)),
                   pl.BlockSpec((BE, D), lambda i: (i, 0))],
        out_shape=[jax.ShapeDtypeStruct((EP, D), F32),
                   jax.ShapeDtypeStruct((EP, D), F32)],
    )(ftS, ftD, g, gt)


def _att_tail_body(last, a0, a1, e0, e1, hn, whr, bhr, ilns, ilnb,
                   wasi, basi, w_x, b_x, watt, o_ref, ft_ref=None):
    denom = e0[0] + e1[0] + 1e-9
    ha = _elu((a0[0] + a1[0]) / denom)
    h = _dot(ha, whr[...]) + bhr[...] + hn[...]
    hn2 = _ln(h, ilns[...], ilnb[...])
    h = hn2 + _elu(_dot(hn2, wasi[...]) + basi[...])
    if last:
        o_ref[...] = _dot(h, w_x[...]) + b_x[...]
    else:
        hn_n = _ln(h, w_x[...], b_x[...])
        o_ref[...] = hn_n
        ft_ref[...] = _dot(hn_n, watt[...])


def _tc_att_tail(last, pA, pE, hn, whr, bhr, ilns, ilnb, wasi, basi,
                 w_x, b_x, watt):
    n_out = 1 if last else 2
    x_spec = _w_spec if last else _v_spec
    out_specs = [_row_spec] * n_out
    out_shape = [jax.ShapeDtypeStruct((NP, D), F32)] * n_out
    return pl.pallas_call(
        functools.partial(_att_tail_body, last),
        grid=(NP // BR,),
        in_specs=_p_specs(D) + _p_specs(D)
        + [_row_spec,
           _w_spec, _v_spec, _v_spec, _v_spec, _w_spec, _v_spec,
           x_spec, _v_spec, _w_spec],
        out_specs=out_specs if n_out > 1 else out_specs[0],
        out_shape=out_shape if n_out > 1 else out_shape[0],
    )(pA, pA, pE, pE, hn, whr, bhr, ilns, ilnb, wasi, basi, w_x, b_x,
      watt)


def _sc_segsum(tab, src2, dst2, zeros128):
    lo = _sc_segsum_half(tab, src2, dst2, zeros128, 0)
    hi = _sc_segsum_half(tab, src2, dst2, zeros128, 1)
    return jnp.concatenate([lo[:, :HN], hi[:, HN:]], axis=1)


def _sc_scatter_rows(vals, dst2, zeros128):
    lo = _sc_scatter_rows_half(vals, dst2, zeros128, 0)
    hi = _sc_scatter_rows_half(vals, dst2, zeros128, 1)
    return jnp.concatenate([lo[:, :HN], hi[:, HN:]], axis=1)


def kernel(x, edge_index, W_pre, b_pre, conv_ln_s, conv_ln_b, W_self,
           W_neigh, b_conv, conv_iln_s, conv_iln_b, W_csi, b_csi, att_ln_s,
           att_ln_b, W_att, W_hr, b_hr, att_iln_s, att_iln_b, W_asi, b_asi,
           W_post, b_post):
    src = edge_index[0]
    dst = edge_index[1]
    pad_idx = (jnp.arange(EP - E, dtype=jnp.int32) % CH) + N
    src_f = jnp.concatenate([src, pad_idx])
    dst_f = jnp.concatenate([dst, pad_idx])
    src2 = src_f.reshape(ER, CH)
    dst2 = dst_f.reshape(ER, CH)
    src2b = src_f.reshape(ER2, CH2)
    dst2b = dst_f.reshape(ER2, CH2)
    x_p = jnp.pad(x, ((0, NP - N), (0, 0)))

    zeros128 = jnp.zeros((CH, D), F32)
    ones256 = jnp.ones((CH2, D), F32)
    g_mat = jnp.asarray(_G_np)
    gt_mat = jnp.asarray(_GT_np)

    def v(a):
        return a.reshape(1, -1)

    pD = _sc_degree(dst2b, ones256, zeros128)
    hn = _tc_pre(x_p, W_pre, v(b_pre), v(conv_ln_s[0]), v(conv_ln_b[0]))

    for i in range(3):
        pA = _sc_segsum(hn, src2b, dst2b, zeros128)
        has_ft = i == 2
        nlns = v(att_ln_s[0]) if has_ft else v(conv_ln_s[i + 1])
        nlnb = v(att_ln_b[0]) if has_ft else v(conv_ln_b[i + 1])
        outs = _tc_conv_tail(has_ft, hn, pA, pD, W_self[i], W_neigh[i],
                             v(b_conv[i]), v(conv_iln_s[i]),
                             v(conv_iln_b[i]), W_csi[i], v(b_csi[i]),
                             nlns, nlnb, W_att[0])
        if has_ft:
            hn, ft = outs
        else:
            hn = outs

    for i in range(3):
        ftS, ftD = _sc_gather2(ft, src2, dst2)
        eew, wft = _tc_att_edge(ftS, ftD, g_mat, gt_mat)
        pA = _sc_scatter_rows(wft, dst2b, zeros128)
        pE = _sc_scatter_rows(eew, dst2b, zeros128)
        last = i == 2
        w_x = W_post if last else v(att_ln_s[i + 1])
        b_x = v(b_post) if last else v(att_ln_b[i + 1])
        watt = W_att[0] if last else W_att[i + 1]
        outs = _tc_att_tail(last, pA, pE, hn, W_hr[i], v(b_hr[i]),
                            v(att_iln_s[i]), v(att_iln_b[i]), W_asi[i],
                            v(b_asi[i]), w_x, b_x, watt)
        if last:
            out = outs
        else:
            hn, ft = outs

    return out[:N]


# speedup vs baseline: 1.2310x; 1.0032x over previous
import functools

import jax
import jax.numpy as jnp
import numpy as np
from jax import lax
from jax.experimental import pallas as pl
from jax.experimental.pallas import tpu as pltpu
from jax.experimental.pallas import tpu_sc as plsc

N = 10000
NP = 10240
HN = NP // 2
JR = 1024
AR = HN + JR
E = 320000
D = 128
NH = 8
NC = 2
NS = 16
NW = NC * NS
CH = 128
RPW = 80
EP = NW * RPW * CH
ER = NW * RPW
CH2 = 128
RPW2 = EP // (NW * CH2)
ER2 = EP // CH2
BR = 1024
BE = 8192
GN = NP // NS
F32 = jnp.float32


@functools.cache
def _mesh():
    return plsc.VectorSubcoreMesh(
        core_axis_name="c", subcore_axis_name="s",
        num_cores=NC, num_subcores=NS)


_G_np = np.zeros((D, NH), np.float32)
for _d in range(D):
    _G_np[_d, _d // 16] = 1.0
_GT_np = np.ascontiguousarray(_G_np.T)


def _redirect(didx, dred, half):
    nrows, ncols = didx.shape

    def body(r, carry):
        for j in range(ncols // 16):
            v = didx[r, pl.ds(j * 16, 16)]
            junk = HN + (v & (JR - 1))
            if half == 0:
                red = jnp.where(v < HN, v, junk)
            else:
                red = jnp.where(v >= HN, v - HN, junk)
            dred[r, pl.ds(j * 16, 16)] = red
        return carry

    lax.fori_loop(0, nrows, body, 0)


def _zero_acc(acc, zbuf, sid):
    for t in range(AR // NS // CH):
        pltpu.sync_copy(zbuf, acc.at[pl.ds(sid * (AR // NS) + t * CH, CH)])


def _export_acc(acc, rows, out, cid, sid, half):
    for t in range(5):
        r0 = sid * (HN // NS) + t * 64
        pltpu.sync_copy(acc.at[pl.ds(r0, 64)], rows)
        pltpu.sync_copy(rows, out.at[cid, pl.ds(half * HN + r0, 64)])


@functools.cache
def _build_segsum(half):
    return pl.kernel(
        functools.partial(_segsum_body, half),
        out_type=jax.ShapeDtypeStruct((NC, NP, D), F32),
        mesh=_mesh(),
        scratch_types=[
            pltpu.VMEM((RPW2, CH2), jnp.int32),
            pltpu.VMEM((RPW2, CH2), jnp.int32),
            pltpu.VMEM((CH2, D), F32),
            pltpu.VMEM((CH2, D), F32),
            pltpu.VMEM((64, D), F32),
            pltpu.VMEM((CH, D), F32),
            pltpu.SemaphoreType.DMA,
            pltpu.SemaphoreType.DMA,
            pltpu.VMEM_SHARED((AR, D), F32),
        ],
    )


def _segsum_body(half, tab, src2, dst2, zeros128, out,
                 sidx, dred, rows0, rows1, erows, zbuf, sem0, sem1, acc):
    cid = lax.axis_index("c")
    sid = lax.axis_index("s")
    w = cid * NS + sid
    pltpu.sync_copy(zeros128, zbuf)
    _zero_acc(acc, zbuf, sid)
    pltpu.sync_copy(src2.at[pl.ds(w * RPW2, RPW2)], sidx)
    pltpu.sync_copy(dst2.at[pl.ds(w * RPW2, RPW2)], dred)
    _redirect(dred, dred, half)
    plsc.subcore_barrier()

    pltpu.async_copy(tab.at[sidx.at[0]], rows0, sem0)

    def body(k, carry):
        g0 = 2 * k
        pltpu.async_copy(tab.at[sidx.at[g0 + 1]], rows1, sem1)
        pltpu.make_async_copy(tab.at[sidx.at[g0]], rows0, sem0).wait()
        pltpu.sync_copy(rows0, acc.at[dred.at[g0]], add=True)

        @pl.when(k < RPW2 // 2 - 1)
        def _():
            pltpu.async_copy(tab.at[sidx.at[g0 + 2]], rows0, sem0)

        pltpu.make_async_copy(tab.at[sidx.at[g0 + 1]], rows1, sem1).wait()
        pltpu.sync_copy(rows1, acc.at[dred.at[g0 + 1]], add=True)
        return carry

    lax.fori_loop(0, RPW2 // 2, body, 0)
    plsc.subcore_barrier()
    _export_acc(acc, erows, out, cid, sid, half)


def _sc_segsum_half(tab, src2, dst2, zeros128, half):
    return _build_segsum(half)(tab, src2, dst2, zeros128)


@functools.cache
def _build_scatter_rows(half):
    return pl.kernel(
        functools.partial(_scatter_rows_body, half),
        out_type=jax.ShapeDtypeStruct((NC, NP, D), F32),
        mesh=_mesh(),
        scratch_types=[
            pltpu.VMEM((RPW2, CH2), jnp.int32),
            pltpu.VMEM((CH2, D), F32),
            pltpu.VMEM((CH2, D), F32),
            pltpu.VMEM((64, D), F32),
            pltpu.VMEM((CH, D), F32),
            pltpu.SemaphoreType.DMA,
            pltpu.SemaphoreType.DMA,
            pltpu.VMEM_SHARED((AR, D), F32),
        ],
    )


def _scatter_rows_body(half, vals, dst2, zeros128, out,
                       dred, rows0, rows1, erows, zbuf, sem0, sem1, acc):
    cid = lax.axis_index("c")
    sid = lax.axis_index("s")
    w = cid * NS + sid
    pltpu.sync_copy(zeros128, zbuf)
    _zero_acc(acc, zbuf, sid)
    pltpu.sync_copy(dst2.at[pl.ds(w * RPW2, RPW2)], dred)
    _redirect(dred, dred, half)
    plsc.subcore_barrier()

    def vsrc(g):
        base = pl.multiple_of((w * RPW2 + g) * CH2, CH2)
        return vals.at[pl.ds(base, CH2)]

    pltpu.async_copy(vsrc(0), rows0, sem0)

    def body(k, carry):
        g0 = 2 * k
        pltpu.async_copy(vsrc(g0 + 1), rows1, sem1)
        pltpu.make_async_copy(vsrc(g0), rows0, sem0).wait()
        pltpu.sync_copy(rows0, acc.at[dred.at[g0]], add=True)

        @pl.when(k < RPW2 // 2 - 1)
        def _():
            pltpu.async_copy(vsrc(g0 + 2), rows0, sem0)

        pltpu.make_async_copy(vsrc(g0 + 1), rows1, sem1).wait()
        pltpu.sync_copy(rows1, acc.at[dred.at[g0 + 1]], add=True)
        return carry

    lax.fori_loop(0, RPW2 // 2, body, 0)
    plsc.subcore_barrier()
    _export_acc(acc, erows, out, cid, sid, half)


def _sc_scatter_rows_half(vals, dst2, zeros128, half):
    return _build_scatter_rows(half)(vals, dst2, zeros128)


@functools.cache
def _build_degree(half):
    return pl.kernel(
        functools.partial(_degree_body, half),
        out_type=jax.ShapeDtypeStruct((NC, NP, D), F32),
        mesh=_mesh(),
        scratch_types=[
            pltpu.VMEM((RPW2, CH2), jnp.int32),
            pltpu.VMEM((CH2, D), F32),
            pltpu.VMEM((64, D), F32),
            pltpu.VMEM((CH, D), F32),
            pltpu.SemaphoreType.DMA,
            pltpu.VMEM_SHARED((AR, D), F32),
        ],
    )


def _degree_body(half, dst2, ones128, zeros128, out,
                 dred, onesb, erows, zbuf, sem0, acc):
    cid = lax.axis_index("c")
    sid = lax.axis_index("s")
    w = cid * NS + sid
    pltpu.sync_copy(zeros128, zbuf)
    _zero_acc(acc, zbuf, sid)
    pltpu.sync_copy(ones128, onesb)
    pltpu.sync_copy(dst2.at[pl.ds(w * RPW2, RPW2)], dred)
    _redirect(dred, dred, half)
    plsc.subcore_barrier()

    def body(k, carry):
        a = pltpu.async_copy(onesb, acc.at[dred.at[2 * k]], sem0, add=True)
        b = pltpu.async_copy(onesb, acc.at[dred.at[2 * k + 1]], sem0,
                             add=True)
        a.wait()
        b.wait()
        return carry

    lax.fori_loop(0, RPW2 // 2, body, 0)
    plsc.subcore_barrier()
    _export_acc(acc, erows, out, cid, sid, half)


def _sc_degree(dst2, ones128, zeros128):
    lo = _build_degree(0)(dst2, ones128, zeros128)
    hi = _build_degree(1)(dst2, ones128, zeros128)
    return jnp.concatenate([lo[:, :HN], hi[:, HN:]], axis=1)


@functools.cache
def _build_gather2():
    return pl.kernel(
        _gather2_body,
        out_type=(jax.ShapeDtypeStruct((EP, D), F32),
                  jax.ShapeDtypeStruct((EP, D), F32)),
        mesh=_mesh(),
        scratch_types=[
            pltpu.VMEM((RPW, CH), jnp.int32),
            pltpu.VMEM((RPW, CH), jnp.int32),
            pltpu.VMEM((CH, D), F32),
            pltpu.VMEM((CH, D), F32),
            pltpu.VMEM((CH, D), F32),
            pltpu.VMEM((CH, D), F32),
            pltpu.SemaphoreType.DMA,
            pltpu.SemaphoreType.DMA,
            pltpu.SemaphoreType.DMA,
            pltpu.SemaphoreType.DMA,
        ],
    )


def _sc_gather2(tab, src2, dst2):
    return _build_gather2()(tab, src2, dst2)


def _gather2_body(tab, src2, dst2, outS, outD, sidx, didx,
                  rS0, rS1, rD0, rD1, semS0, semS1, semD0, semD1):
    cid = lax.axis_index("c")
    sid = lax.axis_index("s")
    w = cid * NS + sid
    pltpu.sync_copy(src2.at[pl.ds(w * RPW, RPW)], sidx)
    pltpu.sync_copy(dst2.at[pl.ds(w * RPW, RPW)], didx)

    def obase(g):
        return pl.multiple_of((w * RPW + g) * CH, CH)

    pltpu.async_copy(tab.at[sidx.at[0]], rS0, semS0)
    pltpu.async_copy(tab.at[didx.at[0]], rD0, semD0)

    def body(k, carry):
        g0 = 2 * k
        pltpu.async_copy(tab.at[sidx.at[g0 + 1]], rS1, semS1)
        pltpu.async_copy(tab.at[didx.at[g0 + 1]], rD1, semD1)
        pltpu.make_async_copy(tab.at[sidx.at[g0]], rS0, semS0).wait()
        pltpu.sync_copy(rS0, outS.at[pl.ds(obase(g0), CH)])
        pltpu.make_async_copy(tab.at[didx.at[g0]], rD0, semD0).wait()
        pltpu.sync_copy(rD0, outD.at[pl.ds(obase(g0), CH)])

        @pl.when(k < RPW // 2 - 1)
        def _():
            pltpu.async_copy(tab.at[sidx.at[g0 + 2]], rS0, semS0)
            pltpu.async_copy(tab.at[didx.at[g0 + 2]], rD0, semD0)

        pltpu.make_async_copy(tab.at[sidx.at[g0 + 1]], rS1, semS1).wait()
        pltpu.sync_copy(rS1, outS.at[pl.ds(obase(g0 + 1), CH)])
        pltpu.make_async_copy(tab.at[didx.at[g0 + 1]], rD1, semD1).wait()
        pltpu.sync_copy(rD1, outD.at[pl.ds(obase(g0 + 1), CH)])
        return carry

    lax.fori_loop(0, RPW // 2, body, 0)


def _ln(x, s, b):
    mu = jnp.mean(x, axis=-1, keepdims=True)
    xc = x - mu
    var = jnp.mean(xc * xc, axis=-1, keepdims=True)
    return xc / jnp.sqrt(var + 1e-5) * s + b


def _elu(x):
    return jnp.where(x > 0, x, jnp.exp(x) - 1.0)


def _dot(a, b):
    return jnp.dot(a, b, preferred_element_type=F32)


_row_spec = pl.BlockSpec((BR, D), lambda i: (i, 0))
_w_spec = pl.BlockSpec((D, D), lambda i: (0, 0))
_v_spec = pl.BlockSpec((1, D), lambda i: (0, 0))


def _p_specs(width):
    return [pl.BlockSpec((1, BR, width), lambda i: (0, i, 0)),
            pl.BlockSpec((1, BR, width), lambda i: (1, i, 0))]


def _pre_body(x, wpre, bpre, s0, b0, hn_ref):
    h = _elu(_dot(x[...], wpre[...]) + bpre[...])
    hn_ref[...] = _ln(h, s0[...], b0[...])


def _tc_pre(x, wpre, bpre, s0, b0):
    return pl.pallas_call(
        _pre_body,
        grid=(NP // BR,),
        in_specs=[_row_spec, _w_spec, _v_spec, _v_spec, _v_spec],
        out_specs=_row_spec,
        out_shape=jax.ShapeDtypeStruct((NP, D), F32),
    )(x, wpre, bpre, s0, b0)


def _conv_tail_body(has_ft, hn, p0, p1, d0, d1, wself, wneigh, bconv,
                    ilns, ilnb, wcsi, bcsi, nlns, nlnb, watt,
                    hn_ref, ft_ref=None):
    deg = jnp.maximum(d0[0][:, :1] + d1[0][:, :1], 1.0)
    neigh = (p0[0] + p1[0]) / deg
    hnv = hn[...]
    h = _dot(hnv, wself[...]) + _dot(neigh, wneigh[...]) + bconv[...] + hnv
    hn2 = _ln(h, ilns[...], ilnb[...])
    h = hn2 + _elu(_dot(hn2, wcsi[...]) + bcsi[...])
    hn_n = _ln(h, nlns[...], nlnb[...])
    hn_ref[...] = hn_n
    if has_ft:
        ft_ref[...] = _dot(hn_n, watt[...])


def _tc_conv_tail(has_ft, hn, pA, pD, wself, wneigh, bconv, ilns, ilnb,
                  wcsi, bcsi, nlns, nlnb, watt):
    n_out = 2 if has_ft else 1
    out_specs = [_row_spec] * n_out
    out_shape = [jax.ShapeDtypeStruct((NP, D), F32)] * n_out
    return pl.pallas_call(
        functools.partial(_conv_tail_body, has_ft),
        grid=(NP // BR,),
        in_specs=[_row_spec] + _p_specs(D) + _p_specs(D)
        + [_w_spec, _w_spec, _v_spec, _v_spec, _v_spec, _w_spec, _v_spec,
           _v_spec, _v_spec, _w_spec],
        out_specs=out_specs if has_ft else out_specs[0],
        out_shape=out_shape if has_ft else out_shape[0],
    )(hn, pA, pA, pD, pD, wself, wneigh, bconv, ilns, ilnb, wcsi, bcsi,
      nlns, nlnb, watt)


def _att_edge_body(ftS, ftD, g_ref, gt_ref, eew_ref, wft_ref):
    fs = ftS[...]
    prod = fs * ftD[...]
    e = _dot(prod, g_ref[...]) * 0.25
    eew = _dot(jnp.exp(e), gt_ref[...])
    eew_ref[...] = eew
    wft_ref[...] = fs * eew


def _tc_att_edge(ftS, ftD, g, gt):
    return pl.pallas_call(
        _att_edge_body,
        grid=(EP // BE,),
        in_specs=[pl.BlockSpec((BE, D), lambda i: (i, 0)),
                  pl.BlockSpec((BE, D), lambda i: (i, 0)),
                  pl.BlockSpec((D, NH), lambda i: (0, 0)),
                  pl.BlockSpec((NH, D), lambda i: (0, 0))],
        out_specs=[pl.BlockSpec((BE, D), lambda i: (i, 0)),
                   pl.BlockSpec((BE, D), lambda i: (i, 0))],
        out_shape=[jax.ShapeDtypeStruct((EP, D), F32),
                   jax.ShapeDtypeStruct((EP, D), F32)],
    )(ftS, ftD, g, gt)


def _att_tail_body(last, a0, a1, e0, e1, hn, whr, bhr, ilns, ilnb,
                   wasi, basi, w_x, b_x, watt, o_ref, ft_ref=None):
    denom = e0[0] + e1[0] + 1e-9
    ha = _elu((a0[0] + a1[0]) / denom)
    h = _dot(ha, whr[...]) + bhr[...] + hn[...]
    hn2 = _ln(h, ilns[...], ilnb[...])
    h = hn2 + _elu(_dot(hn2, wasi[...]) + basi[...])
    if last:
        o_ref[...] = _dot(h, w_x[...]) + b_x[...]
    else:
        hn_n = _ln(h, w_x[...], b_x[...])
        o_ref[...] = hn_n
        ft_ref[...] = _dot(hn_n, watt[...])


def _tc_att_tail(last, pA, pE, hn, whr, bhr, ilns, ilnb, wasi, basi,
                 w_x, b_x, watt):
    n_out = 1 if last else 2
    x_spec = _w_spec if last else _v_spec
    out_specs = [_row_spec] * n_out
    out_shape = [jax.ShapeDtypeStruct((NP, D), F32)] * n_out
    return pl.pallas_call(
        functools.partial(_att_tail_body, last),
        grid=(NP // BR,),
        in_specs=_p_specs(D) + _p_specs(D)
        + [_row_spec,
           _w_spec, _v_spec, _v_spec, _v_spec, _w_spec, _v_spec,
           x_spec, _v_spec, _w_spec],
        out_specs=out_specs if n_out > 1 else out_specs[0],
        out_shape=out_shape if n_out > 1 else out_shape[0],
    )(pA, pA, pE, pE, hn, whr, bhr, ilns, ilnb, wasi, basi, w_x, b_x,
      watt)


def _sc_segsum(tab, src2, dst2, zeros128):
    lo = _sc_segsum_half(tab, src2, dst2, zeros128, 0)
    hi = _sc_segsum_half(tab, src2, dst2, zeros128, 1)
    return jnp.concatenate([lo[:, :HN], hi[:, HN:]], axis=1)


def _sc_scatter_rows(vals, dst2, zeros128):
    lo = _sc_scatter_rows_half(vals, dst2, zeros128, 0)
    hi = _sc_scatter_rows_half(vals, dst2, zeros128, 1)
    return jnp.concatenate([lo[:, :HN], hi[:, HN:]], axis=1)


def kernel(x, edge_index, W_pre, b_pre, conv_ln_s, conv_ln_b, W_self,
           W_neigh, b_conv, conv_iln_s, conv_iln_b, W_csi, b_csi, att_ln_s,
           att_ln_b, W_att, W_hr, b_hr, att_iln_s, att_iln_b, W_asi, b_asi,
           W_post, b_post):
    src = edge_index[0]
    dst = edge_index[1]
    pad_idx = (jnp.arange(EP - E, dtype=jnp.int32) % CH) + N
    src_f = jnp.concatenate([src, pad_idx])
    dst_f = jnp.concatenate([dst, pad_idx])
    src2 = src_f.reshape(ER, CH)
    dst2 = dst_f.reshape(ER, CH)
    src2b = src_f.reshape(ER2, CH2)
    dst2b = dst_f.reshape(ER2, CH2)
    x_p = jnp.pad(x, ((0, NP - N), (0, 0)))

    zeros128 = jnp.zeros((CH, D), F32)
    ones256 = jnp.ones((CH2, D), F32)
    g_mat = jnp.asarray(_G_np)
    gt_mat = jnp.asarray(_GT_np)

    def v(a):
        return a.reshape(1, -1)

    pD = _sc_degree(dst2b, ones256, zeros128)
    hn = _tc_pre(x_p, W_pre, v(b_pre), v(conv_ln_s[0]), v(conv_ln_b[0]))

    for i in range(3):
        pA = _sc_segsum(hn, src2b, dst2b, zeros128)
        has_ft = i == 2
        nlns = v(att_ln_s[0]) if has_ft else v(conv_ln_s[i + 1])
        nlnb = v(att_ln_b[0]) if has_ft else v(conv_ln_b[i + 1])
        outs = _tc_conv_tail(has_ft, hn, pA, pD, W_self[i], W_neigh[i],
                             v(b_conv[i]), v(conv_iln_s[i]),
                             v(conv_iln_b[i]), W_csi[i], v(b_csi[i]),
                             nlns, nlnb, W_att[0])
        if has_ft:
            hn, ft = outs
        else:
            hn = outs

    for i in range(3):
        ftS, ftD = _sc_gather2(ft, src2, dst2)
        eew, wft = _tc_att_edge(ftS, ftD, g_mat, gt_mat)
        pA = _sc_scatter_rows(wft, dst2b, zeros128)
        pE = _sc_scatter_rows(eew, dst2b, zeros128)
        last = i == 2
        w_x = W_post if last else v(att_ln_s[i + 1])
        b_x = v(b_post) if last else v(att_ln_b[i + 1])
        watt = W_att[0] if last else W_att[i + 1]
        outs = _tc_att_tail(last, pA, pE, hn, W_hr[i], v(b_hr[i]),
                            v(att_iln_s[i]), v(att_iln_b[i]), W_asi[i],
                            v(b_asi[i]), w_x, b_x, watt)
        if last:
            out = outs
        else:
            hn, ft = outs

    return out[:N]
